# Initial kernel scaffold; baseline (speedup 1.0000x reference)
import jax
import jax.numpy as jnp
from jax.experimental import pallas as pl


def kernel(inputs, trans):
    raise NotImplementedError("write your pallas kernel here")


# speedup vs baseline: 3.2421x; 3.2421x over previous
import functools

import jax
import jax.numpy as jnp
from jax import lax
from jax.experimental import pallas as pl
from jax.experimental.pallas import tpu as pltpu
from jax.experimental.pallas import tpu_sc as plsc

B = 2
NPB = 1_000_000
TOT = B * NPB
H, W = 37, 120
PIX = H * W
DUMP = B * PIX
TBL = 8896

NC, NS, L = 2, 16, 16
NW = NC * NS

PW = 62_496
GW = PW // L
CHUNK_G = 126
CHUNK_P = CHUNK_G * L
CHUNK_F = CHUNK_P * 4
NCHUNK = GW // CHUNK_G
EX_BASE = NW * PW

_GATHER_DNUMS = lax.GatherDimensionNumbers(
    offset_dims=(), collapsed_slice_dims=(0,), start_index_map=(0,))


def _lane_shift_up(v, idx):
    return lax.gather(v, idx, _GATHER_DNUMS, slice_sizes=(1,),
                      mode=lax.GatherScatterMode.PROMISE_IN_BOUNDS)


def _make_sc_kernel():
    mesh = plsc.VectorSubcoreMesh(core_axis_name="c", subcore_axis_name="s")

    @functools.partial(
        pl.kernel,
        mesh=mesh,
        compiler_params=pltpu.CompilerParams(needs_layout_passes=False),
        out_type=[
            jax.ShapeDtypeStruct((NW, TBL), jnp.int32),
            jax.ShapeDtypeStruct((NW, TBL), jnp.float32),
        ],
        scratch_types=[
            pltpu.VMEM((CHUNK_F,), jnp.float32),
            pltpu.VMEM((TBL,), jnp.int32),
            pltpu.VMEM((TBL,), jnp.float32),
            pltpu.VMEM((16,), jnp.float32),
        ],
    )
    def sc_kernel(in_hbm, trans_hbm, n_out, z_out, in_buf, n_tbl, z_tbl, tv):
        wid = lax.axis_index("c") * NS + lax.axis_index("s")

        lane = lax.iota(jnp.int32, L)
        idx4 = lane * 4
        nxt_idx = jnp.minimum(lane + 1, L - 1).reshape(L, 1)
        last_lane = lane == (L - 1)
        neg1 = jnp.full((L,), -1, jnp.int32)

        def init_body(i, _):
            n_tbl[pl.ds(i * L, L)] = neg1
            return 0
        lax.fori_loop(0, TBL // L, init_body, 0)

        pltpu.sync_copy(trans_hbm, tv)
        tvec = tv[...]
        def coef(j):
            return lax.gather(tvec, jnp.full((L, 1), j, jnp.int32),
                              _GATHER_DNUMS, slice_sizes=(1,),
                              mode=lax.GatherScatterMode.PROMISE_IN_BOUNDS)
        t00, t01, t02, t03 = coef(0), coef(1), coef(2), coef(3)
        t10, t11, t12, t13 = coef(4), coef(5), coef(6), coef(7)
        t20, t21, t22, t23 = coef(8), coef(9), coef(10), coef(11)

        def group(buf, fbase, gid0):
            p0 = plsc.load_gather(buf, [fbase + idx4])
            p1 = plsc.load_gather(buf, [fbase + idx4 + 1])
            p2 = plsc.load_gather(buf, [fbase + idx4 + 2])
            p3 = plsc.load_gather(buf, [fbase + idx4 + 3])
            x_n = t00 * p0 + t01 * p1 + t02 * p2 + t03 * p3
            y_n = t10 * p0 + t11 * p1 + t12 * p2 + t13 * p3
            z = t20 * p0 + t21 * p1 + t22 * p2 + t23 * p3
            x = jnp.clip(x_n / z, 0.0, float(H - 1))
            y = jnp.clip(y_n / z, 0.0, float(W - 1))
            xi = x.astype(jnp.int32)
            yi = y.astype(jnp.int32)
            gid = gid0 + lane
            boff = jnp.where(gid >= NPB, PIX, 0)
            pid = jnp.clip(xi * W + yi, 0, PIX - 1) + boff
            pid = jnp.where(z > 0.0, pid, DUMP)
            key = pid * L + lane
            skey, sgid = plsc.sort_key_val(key, gid)
            _, sz = plsc.sort_key_val(key, z)
            spid = jnp.right_shift(skey, 4)
            is_end = jnp.logical_or(spid != _lane_shift_up(spid, nxt_idx),
                                    last_lane)
            plsc.store_scatter(n_tbl, [spid], sgid, mask=is_end)
            plsc.store_scatter(z_tbl, [spid], sz, mask=is_end)

        def chunk_body(c, _):
            p_base = wid * PW + c * CHUNK_P
            pltpu.sync_copy(in_hbm.at[pl.ds(p_base * 4, CHUNK_F)], in_buf)
            def group_body(g, _):
                group(in_buf, g * (4 * L), p_base + g * L)
                return 0
            lax.fori_loop(0, CHUNK_G, group_body, 0)
            return 0
        lax.fori_loop(0, NCHUNK, chunk_body, 0)

        @pl.when(wid < (TOT - EX_BASE) // L)
        def _():
            ex0 = EX_BASE + wid * L
            pltpu.sync_copy(in_hbm.at[pl.ds(ex0 * 4, 4 * L)],
                            in_buf.at[pl.ds(0, 4 * L)])
            group(in_buf, 0, ex0)

        pltpu.sync_copy(n_tbl, n_out.at[wid])
        pltpu.sync_copy(z_tbl, z_out.at[wid])

    return sc_kernel


def _tc_merge(n_all, z_all):
    def body(n_ref, z_ref, o_ref):
        n = n_ref[...]
        z = z_ref[...]
        bn = jnp.max(n, axis=0, keepdims=True)
        zz = jnp.sum(jnp.where(n == bn, z, 0.0), axis=0, keepdims=True)
        o_ref[...] = jnp.where(bn >= 0, zz, 0.0)

    return pl.pallas_call(
        body,
        out_shape=jax.ShapeDtypeStruct((1, TBL), jnp.float32),
    )(n_all, z_all)


@jax.jit
def kernel(inputs, trans):
    in_flat = inputs.reshape(-1)
    t16 = jnp.pad(trans.reshape(-1), (0, 16 - trans.size))
    n_all, z_all = _make_sc_kernel()(in_flat, t16)
    merged = _tc_merge(n_all, z_all)
    return merged[0, :B * PIX].reshape(B, H, W)


# speedup vs baseline: 15.8176x; 4.8788x over previous
import functools

import jax
import jax.numpy as jnp
from jax import lax
from jax.experimental import pallas as pl
from jax.experimental.pallas import tpu as pltpu
from jax.experimental.pallas import tpu_sc as plsc

B = 2
NPB = 1_000_000
TOT = B * NPB
H, W = 37, 120
PIX = H * W
DUMP = B * PIX
TBL = 8896

NC, NS, L = 2, 16, 16
NW = NC * NS

PW = 62_496
GW = PW // L
CHUNK_G = 126
CHUNK_P = CHUNK_G * L
NCHUNK = GW // CHUNK_G
EX_BASE = NW * PW

_GATHER_DNUMS = lax.GatherDimensionNumbers(
    offset_dims=(), collapsed_slice_dims=(0,), start_index_map=(0,))


def _vgather(v, idx):
    return lax.gather(v, idx, _GATHER_DNUMS, slice_sizes=(1,),
                      mode=lax.GatherScatterMode.PROMISE_IN_BOUNDS)


def _make_sc_kernel():
    mesh = plsc.VectorSubcoreMesh(core_axis_name="c", subcore_axis_name="s")

    @functools.partial(
        pl.kernel,
        mesh=mesh,
        compiler_params=pltpu.CompilerParams(needs_layout_passes=False),
        out_type=[
            jax.ShapeDtypeStruct((NW, TBL), jnp.int32),
            jax.ShapeDtypeStruct((NW, TBL), jnp.float32),
        ],
        scratch_types=[
            pltpu.VMEM((8 * CHUNK_P,), jnp.float32),
            pltpu.VMEM((TBL,), jnp.int32),
            pltpu.VMEM((TBL,), jnp.float32),
            pltpu.VMEM((16,), jnp.float32),
            pltpu.SemaphoreType.DMA,
            pltpu.SemaphoreType.DMA,
        ],
    )
    def sc_kernel(c0, c1, c2, c3, trans_hbm, n_out, z_out,
                  bufs, n_tbl, z_tbl, tv, sem0, sem1):
        comps = (c0, c1, c2, c3)
        sems = (sem0, sem1)
        wid = lax.axis_index("c") * NS + lax.axis_index("s")

        lane = lax.iota(jnp.int32, L)
        nxt_idx = jnp.minimum(lane + 1, L - 1).reshape(L, 1)
        last_lane = lane == (L - 1)
        neg1 = jnp.full((L,), -1, jnp.int32)

        def init_body(i, _):
            n_tbl[pl.ds(i * L, L)] = neg1
            return 0
        lax.fori_loop(0, TBL // L, init_body, 0)

        pltpu.sync_copy(trans_hbm, tv)
        tvec = tv[...]
        def coef(j):
            return _vgather(tvec, jnp.full((L, 1), j, jnp.int32))
        t00, t01, t02, t03 = coef(0), coef(1), coef(2), coef(3)
        t10, t11, t12, t13 = coef(4), coef(5), coef(6), coef(7)
        t20, t21, t22, t23 = coef(8), coef(9), coef(10), coef(11)

        def group(r0, r1, r2, r3, goff, gid0):
            p0 = r0[pl.ds(goff, L)]
            p1 = r1[pl.ds(goff, L)]
            p2 = r2[pl.ds(goff, L)]
            p3 = r3[pl.ds(goff, L)]
            x_n = t00 * p0 + t01 * p1 + t02 * p2 + t03 * p3
            y_n = t10 * p0 + t11 * p1 + t12 * p2 + t13 * p3
            z = t20 * p0 + t21 * p1 + t22 * p2 + t23 * p3
            x = jnp.clip(x_n / z, 0.0, float(H - 1))
            y = jnp.clip(y_n / z, 0.0, float(W - 1))
            xi = x.astype(jnp.int32)
            yi = y.astype(jnp.int32)
            gid = gid0 + lane
            boff = jnp.where(gid >= NPB, PIX, 0)
            pid = jnp.clip(xi * W + yi, 0, PIX - 1) + boff
            pid = jnp.where(z > 0.0, pid, DUMP)
            key = pid * L + lane
            skey, sgid = plsc.sort_key_val(key, gid)
            _, sz = plsc.sort_key_val(key, z)
            spid = jnp.right_shift(skey, 4)
            is_end = jnp.logical_or(spid != _vgather(spid, nxt_idx),
                                    last_lane)
            plsc.store_scatter(n_tbl, [spid], sgid, mask=is_end)
            plsc.store_scatter(z_tbl, [spid], sz, mask=is_end)

        def bslice(par, j, n=CHUNK_P):
            return bufs.at[pl.ds((par * 4 + j) * CHUNK_P, n)]

        def start(c, par):
            off = wid * PW + c * CHUNK_P
            for j in range(4):
                pltpu.async_copy(comps[j].at[pl.ds(off, CHUNK_P)],
                                 bslice(par, j), sems[par])

        def wait(c, par):
            off = wid * PW + c * CHUNK_P
            for j in range(4):
                pltpu.make_async_copy(comps[j].at[pl.ds(off, CHUNK_P)],
                                      bslice(par, j), sems[par]).wait()

        start(0, 0)
        for c in range(NCHUNK):
            par = c % 2
            if c + 1 < NCHUNK:
                start(c + 1, 1 - par)
            wait(c, par)
            r0, r1, r2, r3 = (bslice(par, j) for j in range(4))
            p_base = wid * PW + c * CHUNK_P
            def group_body(g, _):
                group(r0, r1, r2, r3, g * L, p_base + g * L)
                return 0
            lax.fori_loop(0, CHUNK_G, group_body, 0)

        @pl.when(wid < (TOT - EX_BASE) // L)
        def _():
            ex0 = EX_BASE + wid * L
            for j in range(4):
                pltpu.sync_copy(comps[j].at[pl.ds(ex0, L)],
                                bslice(0, j, L))
            r0, r1, r2, r3 = (bslice(0, j, L) for j in range(4))
            group(r0, r1, r2, r3, 0, ex0)

        pltpu.sync_copy(n_tbl, n_out.at[wid])
        pltpu.sync_copy(z_tbl, z_out.at[wid])

    return sc_kernel


def _tc_merge(n_all, z_all):
    def body(n_ref, z_ref, o_ref):
        n = n_ref[...]
        z = z_ref[...]
        bn = jnp.max(n, axis=0, keepdims=True)
        zz = jnp.sum(jnp.where(n == bn, z, 0.0), axis=0, keepdims=True)
        o_ref[...] = jnp.where(bn >= 0, zz, 0.0)

    return pl.pallas_call(
        body,
        out_shape=jax.ShapeDtypeStruct((1, TBL), jnp.float32),
    )(n_all, z_all)


@jax.jit
def kernel(inputs, trans):
    comps = [inputs[:, :, j].reshape(-1) for j in range(4)]
    t16 = jnp.pad(trans.reshape(-1), (0, 16 - trans.size))
    n_all, z_all = _make_sc_kernel()(*comps, t16)
    merged = _tc_merge(n_all, z_all)
    return merged[0, :B * PIX].reshape(B, H, W)
